# baseline (speedup 1.0000x reference)
import functools

import jax
import jax.numpy as jnp
from jax import lax
from jax.experimental import pallas as pl
from jax.experimental.pallas import tpu as pltpu
from jax.experimental.pallas import tpu_sc as plsc

_B, _T, _D = 128, 999, 256
_NC = 2
_TAIL0 = (_T // 8) * 8
_TAIL = _T - _TAIL0
_TR = 64
_LANES = 16
_REP = 16


@functools.partial(
    pl.kernel,
    mesh=plsc.VectorSubcoreMesh(core_axis_name="c", subcore_axis_name="s"),
    out_type=jax.ShapeDtypeStruct((_TAIL, _B, _D), jnp.float32),
    scratch_types=[
        pltpu.VMEM((8, _D), jnp.float32),
        pltpu.VMEM((_REP, _D), jnp.float32),
        pltpu.SemaphoreType.DMA,
    ],
)
def _tail_sc(table_hbm, gtail_hbm, rows_v, rep_v, sem):
    wid = lax.axis_index("s") * _NC + lax.axis_index("c")

    for k in range(_TAIL):
        @pl.when(wid == k)
        def _(k=k):
            pltpu.sync_copy(table_hbm.at[pl.ds(_TAIL0, 8), :], rows_v)
            for c in range(_D // _LANES):
                vec = rows_v[k, pl.ds(c * _LANES, _LANES)]
                for r in range(_REP):
                    rep_v[r, pl.ds(c * _LANES, _LANES)] = vec
            copies = [
                pltpu.async_copy(
                    rep_v, gtail_hbm.at[k].at[pl.ds(r * _REP, _REP), :], sem
                )
                for r in range(_B // _REP)
            ]
            for cp in copies:
                cp.wait()


def _broadcast_tc(w_ref, out_ref):
    out_ref[...] = jnp.broadcast_to(w_ref[...][:, None, :], (_TR, _B, _D))


_broadcast = pl.pallas_call(
    _broadcast_tc,
    grid=((_T + _TR - 1) // _TR,),
    in_specs=[pl.BlockSpec((_TR, _D), lambda i: (i, 0))],
    out_specs=pl.BlockSpec((_TR, _B, _D), lambda i: (i, 0, 0)),
    out_shape=jax.ShapeDtypeStruct((_T, _B, _D), jnp.float32),
)


def kernel(x, col_embed_weight):
    del x
    g_tail = _tail_sc(col_embed_weight)
    out_t = _broadcast(col_embed_weight)
    out_t = lax.dynamic_update_slice(out_t, g_tail, (_TAIL0, 0, 0))
    return jnp.transpose(out_t, (1, 0, 2))


# speedup vs baseline: 1.0143x; 1.0143x over previous
import functools

import jax
import jax.numpy as jnp
from jax import lax
from jax.experimental import pallas as pl
from jax.experimental.pallas import tpu as pltpu
from jax.experimental.pallas import tpu_sc as plsc

_B, _T, _D = 128, 999, 256
_NC = 2
_TAIL0 = (_T // 8) * 8
_TAIL = _T - _TAIL0
_TR = 64
_LANES = 16
_REP = 16


@functools.partial(
    pl.kernel,
    mesh=plsc.VectorSubcoreMesh(core_axis_name="c", subcore_axis_name="s"),
    out_type=jax.ShapeDtypeStruct((_TAIL, _B, _D), jnp.float32),
    scratch_types=[
        pltpu.VMEM((8, _D), jnp.float32),
        pltpu.VMEM((_REP, _D), jnp.float32),
        pltpu.SemaphoreType.DMA,
    ],
)
def _tail_sc(table_hbm, gtail_hbm, rows_v, rep_v, sem):
    wid = lax.axis_index("s") * _NC + lax.axis_index("c")

    @pl.when(wid < _TAIL)
    def _():
        pltpu.sync_copy(table_hbm.at[pl.ds(_TAIL0, 8), :], rows_v)

        def _rep_col(c, carry):
            vec = rows_v[wid, pl.ds(c * _LANES, _LANES)]

            def _rep_row(r, carry2):
                rep_v[r, pl.ds(c * _LANES, _LANES)] = vec
                return carry2

            return lax.fori_loop(0, _REP, _rep_row, carry)

        lax.fori_loop(0, _D // _LANES, _rep_col, 0)

        copies = [
            pltpu.async_copy(
                rep_v, gtail_hbm.at[wid].at[pl.ds(r * _REP, _REP), :], sem
            )
            for r in range(_B // _REP)
        ]
        for cp in copies:
            cp.wait()


def _broadcast_tc(w_ref, out_ref):
    out_ref[...] = jnp.broadcast_to(w_ref[...][:, None, :], (_TR, _B, _D))


_broadcast = pl.pallas_call(
    _broadcast_tc,
    grid=((_T + _TR - 1) // _TR,),
    in_specs=[pl.BlockSpec((_TR, _D), lambda i: (i, 0))],
    out_specs=pl.BlockSpec((_TR, _B, _D), lambda i: (i, 0, 0)),
    out_shape=jax.ShapeDtypeStruct((_T, _B, _D), jnp.float32),
)


def kernel(x, col_embed_weight):
    del x
    g_tail = _tail_sc(col_embed_weight)
    out_t = _broadcast(col_embed_weight)
    out_t = lax.dynamic_update_slice(out_t, g_tail, (_TAIL0, 0, 0))
    return jnp.transpose(out_t, (1, 0, 2))


# speedup vs baseline: 1.0317x; 1.0171x over previous
import functools

import jax
import jax.numpy as jnp
from jax import lax
from jax.experimental import pallas as pl
from jax.experimental.pallas import tpu as pltpu
from jax.experimental.pallas import tpu_sc as plsc

_B, _T, _D = 128, 999, 256
_NC = 2
_TAIL0 = (_T // 8) * 8
_TAIL = _T - _TAIL0
_TR = 64
_LANES = 16
_REP = 16


@functools.partial(
    pl.kernel,
    mesh=plsc.VectorSubcoreMesh(
        core_axis_name="c", subcore_axis_name="s", num_cores=1
    ),
    out_type=jax.ShapeDtypeStruct((_TAIL, _B, _D), jnp.float32),
    scratch_types=[
        pltpu.VMEM((8, _D), jnp.float32),
        pltpu.VMEM((_REP, _D), jnp.float32),
        pltpu.SemaphoreType.DMA,
    ],
)
def _tail_sc(table_hbm, gtail_hbm, rows_v, rep_v, sem):
    wid = lax.axis_index("s")

    @pl.when(wid < _TAIL)
    def _():
        pltpu.sync_copy(table_hbm.at[pl.ds(_TAIL0, 8), :], rows_v)

        def _rep_col(c, carry):
            vec = rows_v[wid, pl.ds(c * _LANES, _LANES)]

            def _rep_row(r, carry2):
                rep_v[r, pl.ds(c * _LANES, _LANES)] = vec
                return carry2

            return lax.fori_loop(0, _REP, _rep_row, carry)

        lax.fori_loop(0, _D // _LANES, _rep_col, 0)

        copies = [
            pltpu.async_copy(
                rep_v, gtail_hbm.at[wid].at[pl.ds(r * _REP, _REP), :], sem
            )
            for r in range(_B // _REP)
        ]
        for cp in copies:
            cp.wait()


def _broadcast_tc(w_ref, out_ref):
    out_ref[...] = jnp.broadcast_to(w_ref[...][:, None, :], (_TR, _B, _D))


_broadcast = pl.pallas_call(
    _broadcast_tc,
    grid=((_T + _TR - 1) // _TR,),
    in_specs=[pl.BlockSpec((_TR, _D), lambda i: (i, 0))],
    out_specs=pl.BlockSpec((_TR, _B, _D), lambda i: (i, 0, 0)),
    out_shape=jax.ShapeDtypeStruct((_T, _B, _D), jnp.float32),
)


def kernel(x, col_embed_weight):
    del x
    g_tail = _tail_sc(col_embed_weight)
    out_t = _broadcast(col_embed_weight)
    out_t = lax.dynamic_update_slice(out_t, g_tail, (_TAIL0, 0, 0))
    return jnp.transpose(out_t, (1, 0, 2))


# speedup vs baseline: 1.0344x; 1.0026x over previous
import functools

import jax
import jax.numpy as jnp
from jax import lax
from jax.experimental import pallas as pl
from jax.experimental.pallas import tpu as pltpu
from jax.experimental.pallas import tpu_sc as plsc

_B, _T, _D = 128, 999, 256
_NC = 2
_TAIL0 = (_T // 8) * 8
_TAIL = _T - _TAIL0
_TR = 64
_LANES = 16
_REP = 16


@functools.partial(
    pl.kernel,
    mesh=plsc.VectorSubcoreMesh(
        core_axis_name="c", subcore_axis_name="s", num_cores=1
    ),
    out_type=jax.ShapeDtypeStruct((_TAIL, _B, _D), jnp.float32),
    scratch_types=[
        pltpu.VMEM((8, _D), jnp.float32),
        pltpu.VMEM((_REP, _D), jnp.float32),
        pltpu.SemaphoreType.DMA,
    ],
)
def _tail_sc(table_hbm, gtail_hbm, rows_v, rep_v, sem):
    wid = lax.axis_index("s")

    @pl.when(wid < _TAIL)
    def _():
        pltpu.sync_copy(table_hbm.at[pl.ds(_TAIL0, 8), :], rows_v)

        def _rep_col(c, carry):
            vec = rows_v[wid, pl.ds(c * _LANES, _LANES)]

            def _rep_row(r, carry2):
                rep_v[r, pl.ds(c * _LANES, _LANES)] = vec
                return carry2

            return lax.fori_loop(0, _REP, _rep_row, carry)

        lax.fori_loop(0, _D // _LANES, _rep_col, 0)

        copies = [
            pltpu.async_copy(
                rep_v, gtail_hbm.at[wid].at[pl.ds(r * _REP, _REP), :], sem
            )
            for r in range(_B // _REP)
        ]
        for cp in copies:
            cp.wait()


def _broadcast_tc(w_ref, out_ref):
    out_ref[...] = jnp.broadcast_to(w_ref[...][:, None, :], (_TR, _B, _D))


_broadcast = pl.pallas_call(
    _broadcast_tc,
    grid=((_T + _TR - 1) // _TR,),
    in_specs=[pl.BlockSpec((_TR, _D), lambda i: (i, 0))],
    out_specs=pl.BlockSpec((_TR, _B, _D), lambda i: (i, 0, 0)),
    out_shape=jax.ShapeDtypeStruct((_T, _B, _D), jnp.float32),
)


def kernel(x, col_embed_weight):
    del x
    out_t = _broadcast(col_embed_weight)
    g_tail = _tail_sc(col_embed_weight)
    out_t = lax.dynamic_update_slice(out_t, g_tail, (_TAIL0, 0, 0))
    return jnp.transpose(out_t, (1, 0, 2))
